# Initial kernel scaffold; baseline (speedup 1.0000x reference)
import jax
import jax.numpy as jnp
from jax.experimental import pallas as pl


def kernel(x, emb):
    raise NotImplementedError("write your pallas kernel here")


# speedup vs baseline: 2.7675x; 2.7675x over previous
import jax
import jax.numpy as jnp
from jax.experimental import pallas as pl

DIM = 1024
SCALE = DIM ** (-0.5)


def _scale_body(emb_ref, out_ref):
    out_ref[...] = emb_ref[...] * SCALE


@jax.jit
def _scaled_copy(emb):
    s, d = emb.shape
    block = 512
    return pl.pallas_call(
        _scale_body,
        grid=(s // block,),
        in_specs=[pl.BlockSpec((block, d), lambda i: (i, 0))],
        out_specs=pl.BlockSpec((block, d), lambda i: (i, 0)),
        out_shape=jax.ShapeDtypeStruct((s, d), emb.dtype),
    )(emb)


def kernel(x, emb):
    seq_len = x.shape[1]
    return _scaled_copy(emb[:seq_len])
